# baseline (speedup 1.0000x reference)
import functools

import jax
import jax.numpy as jnp
from jax import lax
from jax.experimental import pallas as pl
from jax.experimental.pallas import tpu as pltpu
from jax.experimental.pallas import tpu_sc as plsc

_VOCAB = 1000000
_DMODEL = 64
_BATCH = 4096
_SEQ = 200

_L = 128
_NW = 32
_GROUPS = (_BATCH * _SEQ) // _L
_GPW = _GROUPS // _NW
_K = 4
_NCH = _GPW // _K

_mesh = plsc.VectorSubcoreMesh(core_axis_name="c", subcore_axis_name="s")


@functools.partial(
    pl.kernel,
    out_type=jax.ShapeDtypeStruct((_GROUPS, _L, _DMODEL), jnp.float32),
    mesh=_mesh,
    scratch_types=[
        pltpu.VMEM((_K, _L), jnp.int32),
        pltpu.VMEM((_K, _L, _DMODEL), jnp.float32),
        pltpu.SemaphoreType.DMA,
    ],
    compiler_params=pltpu.CompilerParams(use_tc_tiling_on_sc=False),
)
def _emb_lookup(idx_hbm, table_hbm, out_hbm, idx_v, rows_v, gsem):
    wid = lax.axis_index("s") * 2 + lax.axis_index("c")
    base = wid * _GPW

    def chunk(c, _):
        g0 = base + c * _K
        pltpu.sync_copy(idx_hbm.at[pl.ds(g0, _K)], idx_v)
        handles = [
            pltpu.async_copy(table_hbm.at[idx_v.at[j]], rows_v.at[j], gsem)
            for j in range(_K)
        ]
        for h in handles:
            h.wait()
        pltpu.sync_copy(rows_v, out_hbm.at[pl.ds(g0, _K)])
        return ()

    lax.fori_loop(0, _NCH, chunk, ())


def kernel(x, weight):
    idx = x.reshape(_GROUPS, _L).astype(jnp.int32)
    out = _emb_lookup(idx, weight)
    return out.reshape(_BATCH, _SEQ, _DMODEL)


# speedup vs baseline: 1.0328x; 1.0328x over previous
import functools

import jax
import jax.numpy as jnp
from jax import lax
from jax.experimental import pallas as pl
from jax.experimental.pallas import tpu as pltpu
from jax.experimental.pallas import tpu_sc as plsc

_VOCAB = 1000000
_DMODEL = 64
_BATCH = 4096
_SEQ = 200

_L = 128
_NW = 32
_GROUPS = (_BATCH * _SEQ) // _L
_GPW = _GROUPS // _NW
_K = 4
_NCH = _GPW // _K

_mesh = plsc.VectorSubcoreMesh(core_axis_name="c", subcore_axis_name="s")


@functools.partial(
    pl.kernel,
    out_type=jax.ShapeDtypeStruct((_GROUPS, _L, _DMODEL), jnp.float32),
    mesh=_mesh,
    scratch_types=[
        pltpu.VMEM((2, _K, _L), jnp.int32),
        pltpu.VMEM((2, _K, _L, _DMODEL), jnp.float32),
        pltpu.SemaphoreType.DMA,
        pltpu.SemaphoreType.DMA,
        pltpu.SemaphoreType.DMA,
        pltpu.SemaphoreType.DMA,
    ],
    compiler_params=pltpu.CompilerParams(use_tc_tiling_on_sc=False),
)
def _emb_lookup(idx_hbm, table_hbm, out_hbm, idx_v, rows_v, gsem0, gsem1,
                osem0, osem1):
    wid = lax.axis_index("s") * 2 + lax.axis_index("c")
    base = wid * _GPW
    gsems = (gsem0, gsem1)
    osems = (osem0, osem1)

    def fire(c, b):
        pltpu.sync_copy(idx_hbm.at[pl.ds(base + c * _K, _K)], idx_v.at[b])
        for j in range(_K):
            pltpu.async_copy(table_hbm.at[idx_v.at[b].at[j]],
                             rows_v.at[b].at[j], gsems[b])

    def drain_gather(b):
        for j in range(_K):
            pltpu.make_async_copy(table_hbm.at[idx_v.at[b].at[j]],
                                  rows_v.at[b].at[j], gsems[b]).wait()

    def write(c, b):
        pltpu.async_copy(rows_v.at[b], out_hbm.at[pl.ds(base + c * _K, _K)],
                         osems[b])

    def drain_write(c, b):
        pltpu.make_async_copy(rows_v.at[b],
                              out_hbm.at[pl.ds(base + c * _K, _K)],
                              osems[b]).wait()

    fire(0, 0)

    def body(i, _):
        g0 = 2 * i

        @pl.when(i > 0)
        def _():
            drain_write(g0 - 1, 1)

        fire(g0 + 1, 1)
        drain_gather(0)
        write(g0, 0)

        @pl.when(g0 + 2 < _NCH)
        def _():
            drain_write(g0, 0)
            fire(g0 + 2, 0)

        drain_gather(1)
        write(g0 + 1, 1)
        return ()

    lax.fori_loop(0, _NCH // 2, body, ())
    drain_write(_NCH - 2, 0)
    drain_write(_NCH - 1, 1)


def kernel(x, weight):
    idx = x.reshape(_GROUPS, _L).astype(jnp.int32)
    out = _emb_lookup(idx, weight)
    return out.reshape(_BATCH, _SEQ, _DMODEL)


# speedup vs baseline: 1.0339x; 1.0011x over previous
import functools

import jax
import jax.numpy as jnp
from jax import lax
from jax.experimental import pallas as pl
from jax.experimental.pallas import tpu as pltpu
from jax.experimental.pallas import tpu_sc as plsc

_VOCAB = 1000000
_DMODEL = 64
_BATCH = 4096
_SEQ = 200

_B = _BATCH * _SEQ
_NW = 32
_BPW = _B // _NW
_C = 512
_NCH = _BPW // _C

_mesh = plsc.VectorSubcoreMesh(core_axis_name="c", subcore_axis_name="s")


@functools.partial(
    pl.kernel,
    out_type=jax.ShapeDtypeStruct((_B, _DMODEL), jnp.float32),
    mesh=_mesh,
    scratch_types=[
        pltpu.VMEM((2, _C), jnp.int32),
        pltpu.VMEM((2, _C, _DMODEL), jnp.float32),
        pltpu.SemaphoreType.DMA,
        pltpu.SemaphoreType.DMA,
        pltpu.SemaphoreType.DMA,
        pltpu.SemaphoreType.DMA,
    ],
    compiler_params=pltpu.CompilerParams(use_tc_tiling_on_sc=False),
)
def _emb_lookup(idx_hbm, table_hbm, out_hbm, idx_v, rows_v, gsem0, gsem1,
                osem0, osem1):
    wid = lax.axis_index("s") * 2 + lax.axis_index("c")
    base = wid * _BPW
    gsems = (gsem0, gsem1)
    osems = (osem0, osem1)

    def fire(c, b):
        pltpu.sync_copy(idx_hbm.at[pl.ds(base + c * _C, _C)], idx_v.at[b])
        pltpu.async_copy(table_hbm.at[idx_v.at[b]], rows_v.at[b], gsems[b])

    def drain_gather(b):
        pltpu.make_async_copy(table_hbm.at[idx_v.at[b]], rows_v.at[b],
                              gsems[b]).wait()

    def write(c, b):
        pltpu.async_copy(rows_v.at[b], out_hbm.at[pl.ds(base + c * _C, _C)],
                         osems[b])

    def drain_write(c, b):
        pltpu.make_async_copy(rows_v.at[b],
                              out_hbm.at[pl.ds(base + c * _C, _C)],
                              osems[b]).wait()

    fire(0, 0)

    def body(i, _):
        g0 = 2 * i

        @pl.when(i > 0)
        def _():
            drain_write(g0 - 1, 1)

        fire(g0 + 1, 1)
        drain_gather(0)
        write(g0, 0)

        @pl.when(g0 + 2 < _NCH)
        def _():
            drain_write(g0, 0)
            fire(g0 + 2, 0)

        drain_gather(1)
        write(g0 + 1, 1)
        return ()

    lax.fori_loop(0, _NCH // 2, body, ())
    drain_write(_NCH - 2, 0)
    drain_write(_NCH - 1, 1)


def kernel(x, weight):
    idx = x.reshape(_B).astype(jnp.int32)
    out = _emb_lookup(idx, weight)
    return out.reshape(_BATCH, _SEQ, _DMODEL)


# speedup vs baseline: 1.0371x; 1.0031x over previous
import functools

import jax
import jax.numpy as jnp
from jax import lax
from jax.experimental import pallas as pl
from jax.experimental.pallas import tpu as pltpu
from jax.experimental.pallas import tpu_sc as plsc

_VOCAB = 1000000
_DMODEL = 64
_BATCH = 4096
_SEQ = 200

_B = _BATCH * _SEQ
_NW = 32
_BPW = _B // _NW
_C = 512
_NCH = _BPW // _C

_mesh = plsc.VectorSubcoreMesh(core_axis_name="c", subcore_axis_name="s")


@functools.partial(
    pl.kernel,
    out_type=jax.ShapeDtypeStruct((_B, _DMODEL), jnp.float32),
    mesh=_mesh,
    scratch_types=[
        pltpu.VMEM((2, _C), jnp.int32),
        pltpu.VMEM((2, _C, _DMODEL), jnp.float32),
        pltpu.SemaphoreType.DMA,
        pltpu.SemaphoreType.DMA,
        pltpu.SemaphoreType.DMA,
        pltpu.SemaphoreType.DMA,
    ],
    compiler_params=pltpu.CompilerParams(use_tc_tiling_on_sc=False),
)
def _emb_lookup(idx_hbm, table_hbm, out_hbm, idx_v, rows_v, gsem0, gsem1,
                osem0, osem1):
    wid = lax.axis_index("s") * 2 + lax.axis_index("c")
    base = wid * _BPW
    gsems = (gsem0, gsem1)
    osems = (osem0, osem1)

    def fire(c, b):
        pltpu.sync_copy(idx_hbm.at[pl.ds(base + c * _C, _C)], idx_v.at[b])
        for j in range(_C // 16):
            iv = idx_v[b, pl.ds(16 * j, 16)]
            pltpu.async_copy(table_hbm.at[iv],
                             rows_v.at[b].at[pl.ds(16 * j, 16)], gsems[b])

    def drain_gather(b):
        pltpu.make_async_copy(table_hbm.at[idx_v.at[b]], rows_v.at[b],
                              gsems[b]).wait()

    def write(c, b):
        pltpu.async_copy(rows_v.at[b], out_hbm.at[pl.ds(base + c * _C, _C)],
                         osems[b])

    def drain_write(c, b):
        pltpu.make_async_copy(rows_v.at[b],
                              out_hbm.at[pl.ds(base + c * _C, _C)],
                              osems[b]).wait()

    fire(0, 0)

    def body(i, _):
        g0 = 2 * i

        @pl.when(i > 0)
        def _():
            drain_write(g0 - 1, 1)

        fire(g0 + 1, 1)
        drain_gather(0)
        write(g0, 0)

        @pl.when(g0 + 2 < _NCH)
        def _():
            drain_write(g0, 0)
            fire(g0 + 2, 0)

        drain_gather(1)
        write(g0 + 1, 1)
        return ()

    lax.fori_loop(0, _NCH // 2, body, ())
    drain_write(_NCH - 2, 0)
    drain_write(_NCH - 1, 1)


def kernel(x, weight):
    idx = x.reshape(_B).astype(jnp.int32)
    out = _emb_lookup(idx, weight)
    return out.reshape(_BATCH, _SEQ, _DMODEL)


# speedup vs baseline: 1.0423x; 1.0050x over previous
import functools

import jax
import jax.numpy as jnp
from jax import lax
from jax.experimental import pallas as pl
from jax.experimental.pallas import tpu as pltpu
from jax.experimental.pallas import tpu_sc as plsc

_VOCAB = 1000000
_DMODEL = 64
_BATCH = 4096
_SEQ = 200

_B = _BATCH * _SEQ
_NW = 32
_BPW = _B // _NW
_C = 320
_NCH = _BPW // _C
_NBUF = 4

_mesh = plsc.VectorSubcoreMesh(core_axis_name="c", subcore_axis_name="s")


@functools.partial(
    pl.kernel,
    out_type=jax.ShapeDtypeStruct((_B, _DMODEL), jnp.float32),
    mesh=_mesh,
    scratch_types=[
        pltpu.VMEM((_BPW,), jnp.int32),
        pltpu.VMEM((_NBUF, _C, _DMODEL), jnp.float32),
        [pltpu.SemaphoreType.DMA] * _NBUF,
        [pltpu.SemaphoreType.DMA] * _NBUF,
    ],
    compiler_params=pltpu.CompilerParams(use_tc_tiling_on_sc=False),
)
def _emb_lookup(idx_hbm, table_hbm, out_hbm, idx_v, rows_v, gsems, osems):
    wid = lax.axis_index("s") * 2 + lax.axis_index("c")
    base = wid * _BPW

    pltpu.sync_copy(idx_hbm.at[pl.ds(base, _BPW)], idx_v)

    def fire_gather(c, b):
        pltpu.async_copy(table_hbm.at[idx_v.at[pl.ds(c * _C, _C)]],
                         rows_v.at[b], gsems[b])

    def drain_gather(c, b):
        pltpu.make_async_copy(table_hbm.at[idx_v.at[pl.ds(c * _C, _C)]],
                              rows_v.at[b], gsems[b]).wait()

    def write(c, b):
        pltpu.async_copy(rows_v.at[b], out_hbm.at[pl.ds(base + c * _C, _C)],
                         osems[b])

    def drain_write(c, b):
        pltpu.make_async_copy(rows_v.at[b],
                              out_hbm.at[pl.ds(base + c * _C, _C)],
                              osems[b]).wait()

    fire_gather(0, 0)
    fire_gather(1, 1)

    def body(i, _):
        for u in range(_NBUF):
            c = _NBUF * i + u
            b = u
            b2 = (u + 2) % _NBUF
            drain_gather(c, b)
            write(c, b)

            @pl.when(c + 2 < _NCH)
            def _():
                @pl.when(c >= 2)
                def _():
                    drain_write(c - 2, b2)
                fire_gather(c + 2, b2)

        return ()

    lax.fori_loop(0, _NCH // _NBUF, body, ())
    for c in range(_NCH - 4, _NCH):
        drain_write(c, c % _NBUF)


def kernel(x, weight):
    idx = x.reshape(_B).astype(jnp.int32)
    out = _emb_lookup(idx, weight)
    return out.reshape(_BATCH, _SEQ, _DMODEL)


# speedup vs baseline: 1.0443x; 1.0020x over previous
import functools

import jax
import jax.numpy as jnp
from jax import lax
from jax.experimental import pallas as pl
from jax.experimental.pallas import tpu as pltpu
from jax.experimental.pallas import tpu_sc as plsc

_VOCAB = 1000000
_DMODEL = 64
_BATCH = 4096
_SEQ = 200

_B = _BATCH * _SEQ
_NW = 32
_BPW = _B // _NW
_C = 400
_NCH = _BPW // _C
_NBUF = 4

_mesh = plsc.VectorSubcoreMesh(core_axis_name="c", subcore_axis_name="s")


@functools.partial(
    pl.kernel,
    out_type=jax.ShapeDtypeStruct((_B, _DMODEL), jnp.float32),
    mesh=_mesh,
    scratch_types=[
        pltpu.VMEM((_BPW,), jnp.int32),
        pltpu.VMEM((_NBUF, _C, _DMODEL), jnp.float32),
        [pltpu.SemaphoreType.DMA] * _NBUF,
        [pltpu.SemaphoreType.DMA] * _NBUF,
    ],
    compiler_params=pltpu.CompilerParams(use_tc_tiling_on_sc=False),
)
def _emb_lookup(idx_hbm, table_hbm, out_hbm, idx_v, rows_v, gsems, osems):
    wid = lax.axis_index("s") * 2 + lax.axis_index("c")
    base = wid * _BPW

    pltpu.sync_copy(idx_hbm.at[pl.ds(base, _BPW)], idx_v)

    def fire_gather(c, b):
        pltpu.async_copy(table_hbm.at[idx_v.at[pl.ds(c * _C, _C)]],
                         rows_v.at[b], gsems[b])

    def drain_gather(c, b):
        pltpu.make_async_copy(table_hbm.at[idx_v.at[pl.ds(c * _C, _C)]],
                              rows_v.at[b], gsems[b]).wait()

    def write(c, b):
        pltpu.async_copy(rows_v.at[b], out_hbm.at[pl.ds(base + c * _C, _C)],
                         osems[b])

    def drain_write(c, b):
        pltpu.make_async_copy(rows_v.at[b],
                              out_hbm.at[pl.ds(base + c * _C, _C)],
                              osems[b]).wait()

    fire_gather(0, 0)
    fire_gather(1, 1)

    def body(i, _):
        for u in range(_NBUF):
            c = _NBUF * i + u
            b = u
            b2 = (u + 2) % _NBUF
            drain_gather(c, b)
            write(c, b)

            @pl.when(c + 2 < _NCH)
            def _():
                @pl.when(c >= 2)
                def _():
                    drain_write(c - 2, b2)
                fire_gather(c + 2, b2)

        return ()

    lax.fori_loop(0, _NCH // _NBUF, body, ())
    for c in range(_NCH - 4, _NCH):
        drain_write(c, c % _NBUF)


def kernel(x, weight):
    idx = x.reshape(_B).astype(jnp.int32)
    out = _emb_lookup(idx, weight)
    return out.reshape(_BATCH, _SEQ, _DMODEL)
